# baseline (speedup 1.0000x reference)
import jax
import jax.numpy as jnp
from jax.experimental import pallas as pl
from jax.experimental.pallas import tpu as pltpu

_B, _C, _L = 4096, 16, 512
_NUM = _B // 2
_ROWS = _B * _C
_BLK = 7296
_NBLK = -(-_ROWS // _BLK)
_W = _BLK // 32
_IDS_R, _IDS_C = 16, 128


def _body(ids_ref, x_ref, o_ref):
    g = pl.program_id(0)
    wcol = g * _W + jax.lax.broadcasted_iota(jnp.int32, (_W, _IDS_C), 0)
    acc = jnp.zeros((_W, _IDS_C), jnp.int32)
    for c in range(_IDS_R):
        idv = ids_ref[c, :]
        idw = (idv >> 5)[None, :]
        bit = (1 << (idv & 31))[None, :]
        acc = acc | jnp.where(wcol == idw, bit, 0)
    r = acc
    s = _IDS_C // 2
    while s >= 1:
        r = jax.lax.slice_in_dim(r, 0, s, axis=1) | jax.lax.slice_in_dim(r, s, 2 * s, axis=1)
        s //= 2
    bits = jax.lax.broadcasted_iota(jnp.int32, (_W, 32, 1), 1)
    mute = (r.reshape(_W, 1, 1) >> bits) & 1
    o_ref[...] = jnp.where(mute != 0, 0.0, x_ref[...])


def kernel(X):
    B, C, L = X.shape
    k = jax.random.key(42)
    k1, k2 = jax.random.split(k)
    channel = jax.random.randint(k1, (_NUM,), 0, C)
    indices = jax.random.randint(k2, (_NUM,), 0, B)
    flat_ids = (indices * C + channel).reshape(_IDS_R, _IDS_C)
    X3 = X.reshape(_ROWS // 32, 32, _L)
    out = pl.pallas_call(
        _body,
        grid=(_NBLK,),
        in_specs=[
            pl.BlockSpec((_IDS_R, _IDS_C), lambda g: (0, 0)),
            pl.BlockSpec((_W, 32, _L), lambda g: (g, 0, 0)),
        ],
        out_specs=pl.BlockSpec((_W, 32, _L), lambda g: (g, 0, 0)),
        out_shape=jax.ShapeDtypeStruct((_ROWS // 32, 32, _L), X.dtype),
        compiler_params=pltpu.CompilerParams(vmem_limit_bytes=63 * 1024 * 1024),
    )(flat_ids, X3)
    return out.reshape(B, C, L), indices
